# baseline (speedup 1.0000x reference)
import functools

import jax
import jax.numpy as jnp
from jax import lax
from jax.experimental import pallas as pl
from jax.experimental.pallas import tpu as pltpu
from jax.experimental.pallas import tpu_sc as plsc

NC = 21
NOBJ = 1000
NDET = 20000
OP = 1000
P = 1408
DP = 20480
NW = 32
DCH = DP // NW
RS = P + 1
LP = 960
NEGF = -3.4e38


def _prep_body(glr_ref, gll_ref, gbt_ref, dl_ref,
               sx1_ref, sy1_ref, sx2_ref, sy2_ref,
               dpk_ref, nmat_ref):
    glr = glr_ref[...]
    gll = gll_ref[...]
    gbt = gbt_ref[...]
    dl = dl_ref[...]

    base = jnp.int32(0)
    amap = jnp.zeros((1, OP), jnp.int32)
    dbase = jnp.zeros(dl.shape, jnp.int32)
    dcnt = jnp.zeros(dl.shape, jnp.int32)
    nmat = jnp.int32(0)
    for c in range(NC):
        cmask = gll == c
        cnt = jnp.sum(cmask.astype(jnp.int32))
        ng = (cnt + 15) // 16
        amap = jnp.where(cmask, base, amap)
        dmask = dl == c
        hist = jnp.sum(dmask.astype(jnp.int32))
        dbase = jnp.where(dmask, base, dbase)
        dcnt = jnp.where(dmask, cnt, dcnt)
        nmat = nmat + jnp.where(hist > 0, cnt, 0)
        base = base + 16 * ng

    ir = lax.broadcasted_iota(jnp.int32, (OP, OP), 0)
    il = lax.broadcasted_iota(jnp.int32, (OP, OP), 1)
    same = glr == gll
    before = ir < il
    rank = jnp.sum((same & before).astype(jnp.int32), axis=0, keepdims=True)

    pos = amap + rank

    rows = lax.broadcasted_iota(jnp.int32, (P, OP), 0)
    oh = rows == pos
    zero = jnp.float32(0.0)
    sx1_ref[...] = jnp.sum(jnp.where(oh, gbt[0:1, :], zero), axis=1, keepdims=True)
    sy1_ref[...] = jnp.sum(jnp.where(oh, gbt[1:2, :], zero), axis=1, keepdims=True)
    sx2_ref[...] = jnp.sum(jnp.where(oh, gbt[2:3, :], zero), axis=1, keepdims=True)
    sy2_ref[...] = jnp.sum(jnp.where(oh, gbt[3:4, :], zero), axis=1, keepdims=True)
    dpk_ref[...] = (dbase << 11) | dcnt
    nmat_ref[0, 0] = nmat.astype(jnp.float32)


def _sc_body(sx1_h, sy1_h, sx2_h, sy2_h,
             dbf_h, dlab_h, dpk_h,
             out_h,
             sx1, sy1, sx2, sy2, vbox, vlab, vpk,
             lx1, ly1, lx2, ly2, lpk, cnthist, basearr, best2, best, sem):
    wid = lax.axis_index("s") * 2 + lax.axis_index("c")
    dlo = jnp.minimum(wid * DCH, NDET - DCH)

    descs = [
        pltpu.async_copy(sx1_h, sx1, sem),
        pltpu.async_copy(sy1_h, sy1, sem),
        pltpu.async_copy(sx2_h, sx2, sem),
        pltpu.async_copy(sy2_h, sy2, sem),
        pltpu.async_copy(dbf_h.at[pl.ds(dlo * 4, DCH * 4)], vbox, sem),
        pltpu.async_copy(dlab_h.at[pl.ds(dlo, DCH)], vlab, sem),
        pltpu.async_copy(dpk_h.at[pl.ds(dlo, DCH)], vpk, sem),
    ]

    zeros16 = jnp.zeros((16,), jnp.float32)
    zeros16i = jnp.zeros((16,), jnp.int32)

    @plsc.parallel_loop(0, 16 * RS // 16, 1, unroll=4)
    def zbody(i):
        best2[pl.ds(i * 16, 16)] = zeros16

    @plsc.parallel_loop(0, LP // 16, 1, unroll=4)
    def zlbody(i):
        lpk[pl.ds(i * 16, 16)] = zeros16i

    cnthist[pl.ds(0, 16)] = zeros16i
    cnthist[pl.ds(16, 16)] = zeros16i

    for d in descs:
        d.wait()

    def h1body(g, carry):
        lv = vlab[pl.ds(g * 16, 16)]
        scan, lmask = plsc.scan_count(lv)
        cur = plsc.load_gather(cnthist, [lv])
        plsc.store_scatter(cnthist, [lv], cur + scan, mask=lmask)
        return carry
    lax.fori_loop(0, DCH // 16, h1body, 0)

    h0 = cnthist[pl.ds(0, 16)]
    h1 = cnthist[pl.ds(16, 16)]
    ha0 = ((h0 + 15) >> 4) << 4
    ha1 = ((h1 + 15) >> 4) << 4
    e0 = plsc.cumsum(ha0) - ha0
    tot0 = jnp.sum(ha0)
    e1 = plsc.cumsum(ha1) - ha1 + tot0
    basearr[pl.ds(0, 16)] = e0
    basearr[pl.ds(16, 16)] = e1

    cnthist[pl.ds(0, 16)] = zeros16i
    cnthist[pl.ds(16, 16)] = zeros16i

    lane4 = lax.broadcasted_iota(jnp.int32, (16,), 0) * 4

    def s2body(g, carry):
        o = g * 16
        lv = vlab[pl.ds(o, 16)]
        scan, lmask = plsc.scan_count(lv)
        cur = plsc.load_gather(cnthist, [lv])
        bse = plsc.load_gather(basearr, [lv])
        pos = bse + cur + scan - 1
        i4 = lane4 + (o * 4)
        plsc.store_scatter(lx1, [pos], plsc.load_gather(vbox, [i4]))
        plsc.store_scatter(ly1, [pos], plsc.load_gather(vbox, [i4 + 1]))
        plsc.store_scatter(lx2, [pos], plsc.load_gather(vbox, [i4 + 2]))
        plsc.store_scatter(ly2, [pos], plsc.load_gather(vbox, [i4 + 3]))
        plsc.store_scatter(lpk, [pos], vpk[pl.ds(o, 16)])
        plsc.store_scatter(cnthist, [lv], cur + scan, mask=lmask)
        return carry
    lax.fori_loop(0, DCH // 16, s2body, 0)

    lanerow = lax.broadcasted_iota(jnp.int32, (16,), 0) * RS
    k16 = [jnp.full((16,), k, jnp.int32) for k in range(16)]

    def gbody(g, carry):
        b16 = g * 16
        cx1 = lx1[pl.ds(b16, 16)]
        cy1 = ly1[pl.ds(b16, 16)]
        cx2 = lx2[pl.ds(b16, 16)]
        cy2 = ly2[pl.ds(b16, 16)]
        pk = lpk[pl.ds(b16, 16)]
        cnv = pk & 2047
        da = (cx2 - cx1) * (cy2 - cy1)
        pks = jnp.max(pk)
        bks = pks >> 11
        cs = pks & 2047
        nt = (cs + 15) >> 4

        @plsc.parallel_loop(0, nt, 1, unroll=1)
        def wbody(tt):
            o = bks + tt * 16
            wx1 = sx1[pl.ds(o, 16)]
            wy1 = sy1[pl.ds(o, 16)]
            wx2 = sx2[pl.ds(o, 16)]
            wy2 = sy2[pl.ds(o, 16)]
            for k in range(16):
                t = tt * 16 + k
                m = cnv > t
                g1x = jnp.take(wx1, k16[k])
                g1y = jnp.take(wy1, k16[k])
                g2x = jnp.take(wx2, k16[k])
                g2y = jnp.take(wy2, k16[k])
                ga = (g2x - g1x) * (g2y - g1y)
                lox = jnp.maximum(g1x, cx1)
                loy = jnp.maximum(g1y, cy1)
                hix = jnp.minimum(g2x, cx2)
                hiy = jnp.minimum(g2y, cy2)
                ww = jnp.maximum(hix - lox, 0.0)
                hh = jnp.maximum(hiy - loy, 0.0)
                inter = ww * hh
                uni = ga + da - inter
                iou = inter / uni
                bidx = lanerow + (o + k)
                cur = plsc.load_gather(best2, [bidx], mask=m)
                plsc.store_scatter(best2, [bidx], jnp.maximum(cur, iou), mask=m)
        return carry

    lax.fori_loop(0, LP // 16, gbody, 0)

    @plsc.parallel_loop(0, P // 16, 1, unroll=2)
    def rbody(i):
        o = i * 16
        acc = best2[pl.ds(o, 16)]
        for r in range(1, 16):
            acc = jnp.maximum(acc, best2[pl.ds(r * RS + o, 16)])
        best[pl.ds(o, 16)] = acc

    pltpu.sync_copy(best, out_h.at[wid])


def _fin_body(parts_ref, nmat_ref, out_ref):
    parts = parts_ref[...]
    best = jnp.max(parts, axis=0)
    s = jnp.sum(best)
    n = nmat_ref[0, 0]
    out_ref[0, 0] = (n - s) / n


def kernel(det_boxes, det_scores, det_labels, boxes, labels):
    del det_scores
    db = det_boxes[0]
    dl = det_labels[0].astype(jnp.int32)
    gb = boxes[0]
    gl = labels[0].astype(jnp.int32)

    gbt_gt = gb.T

    dlp = jnp.full((DP,), -1, jnp.int32).at[:NDET].set(dl)
    dbf = db.reshape(4 * NDET)

    _vmem = pl.BlockSpec(memory_space=pltpu.VMEM)
    _smem = pl.BlockSpec(memory_space=pltpu.SMEM)
    prep = pl.pallas_call(
        _prep_body,
        out_shape=[
            jax.ShapeDtypeStruct((P, 1), jnp.float32),
            jax.ShapeDtypeStruct((P, 1), jnp.float32),
            jax.ShapeDtypeStruct((P, 1), jnp.float32),
            jax.ShapeDtypeStruct((P, 1), jnp.float32),
            jax.ShapeDtypeStruct((DP // 128, 128), jnp.int32),
            jax.ShapeDtypeStruct((1, 1), jnp.float32),
        ],
        out_specs=[_vmem] * 5 + [_smem],
    )(gl.reshape(OP, 1), gl.reshape(1, OP), gbt_gt,
      dlp.reshape(DP // 128, 128))
    sx1, sy1, sx2, sy2, dpk, nmat = prep

    mesh = plsc.VectorSubcoreMesh(core_axis_name="c", subcore_axis_name="s")
    sc_main = functools.partial(
        pl.kernel,
        out_type=jax.ShapeDtypeStruct((NW, P), jnp.float32),
        mesh=mesh,
        compiler_params=pltpu.CompilerParams(needs_layout_passes=False),
        scratch_types=[
            pltpu.VMEM((P,), jnp.float32),
            pltpu.VMEM((P,), jnp.float32),
            pltpu.VMEM((P,), jnp.float32),
            pltpu.VMEM((P,), jnp.float32),
            pltpu.VMEM((DCH * 4,), jnp.float32),
            pltpu.VMEM((DCH,), jnp.int32),
            pltpu.VMEM((DCH,), jnp.int32),
            pltpu.VMEM((LP,), jnp.float32),
            pltpu.VMEM((LP,), jnp.float32),
            pltpu.VMEM((LP,), jnp.float32),
            pltpu.VMEM((LP,), jnp.float32),
            pltpu.VMEM((LP,), jnp.int32),
            pltpu.VMEM((32,), jnp.int32),
            pltpu.VMEM((32,), jnp.int32),
            pltpu.VMEM((16 * RS,), jnp.float32),
            pltpu.VMEM((P,), jnp.float32),
            pltpu.SemaphoreType.DMA,
        ],
    )(_sc_body)
    parts = sc_main(sx1.reshape(P), sy1.reshape(P), sx2.reshape(P), sy2.reshape(P),
                    dbf, dl, dpk.reshape(DP))

    loss = pl.pallas_call(
        _fin_body,
        out_shape=jax.ShapeDtypeStruct((1, 1), jnp.float32),
        in_specs=[_vmem, _smem],
        out_specs=_smem,
    )(parts, nmat)
    return loss.reshape(())


# speedup vs baseline: 1.3611x; 1.3611x over previous
import functools

import jax
import jax.numpy as jnp
from jax import lax
from jax.experimental import pallas as pl
from jax.experimental.pallas import tpu as pltpu
from jax.experimental.pallas import tpu_sc as plsc

NC = 21
NOBJ = 1000
NDET = 20000
OP = 1000
P = 1408
DP = 20480
NW = 32
DCH = DP // NW
RS = P + 1
LP = 960
NEGF = -3.4e38


def _prep_body(glr_ref, gll_ref, gbt_ref, dl_ref,
               sx1_ref, sy1_ref, sx2_ref, sy2_ref,
               dpk_ref, nmat_ref):
    glr = glr_ref[...]
    gll = gll_ref[...]
    gbt = gbt_ref[...]
    dl = dl_ref[...]

    base = jnp.int32(0)
    amap = jnp.zeros((1, OP), jnp.int32)
    dbase = jnp.zeros(dl.shape, jnp.int32)
    dcnt = jnp.zeros(dl.shape, jnp.int32)
    nmat = jnp.int32(0)
    for c in range(NC):
        cmask = gll == c
        cnt = jnp.sum(cmask.astype(jnp.int32))
        ng = (cnt + 15) // 16
        amap = jnp.where(cmask, base, amap)
        dmask = dl == c
        hist = jnp.sum(dmask.astype(jnp.int32))
        dbase = jnp.where(dmask, base, dbase)
        dcnt = jnp.where(dmask, cnt, dcnt)
        nmat = nmat + jnp.where(hist > 0, cnt, 0)
        base = base + 16 * ng

    ir = lax.broadcasted_iota(jnp.int32, (OP, OP), 0)
    il = lax.broadcasted_iota(jnp.int32, (OP, OP), 1)
    same = glr == gll
    before = ir < il
    rank = jnp.sum((same & before).astype(jnp.int32), axis=0, keepdims=True)

    pos = amap + rank

    rows = lax.broadcasted_iota(jnp.int32, (P, OP), 0)
    oh = rows == pos
    zero = jnp.float32(0.0)
    sx1_ref[...] = jnp.sum(jnp.where(oh, gbt[0:1, :], zero), axis=1, keepdims=True)
    sy1_ref[...] = jnp.sum(jnp.where(oh, gbt[1:2, :], zero), axis=1, keepdims=True)
    sx2_ref[...] = jnp.sum(jnp.where(oh, gbt[2:3, :], zero), axis=1, keepdims=True)
    sy2_ref[...] = jnp.sum(jnp.where(oh, gbt[3:4, :], zero), axis=1, keepdims=True)
    dpk_ref[...] = (dbase << 11) | dcnt
    nmat_ref[0, 0] = nmat.astype(jnp.float32)


def _sc_body(sx1_h, sy1_h, sx2_h, sy2_h,
             dbt_h, dlab_h, dpk_h,
             out_h,
             sx1, sy1, sx2, sy2, vx1, vy1, vx2, vy2, vlab, vpk,
             lx1, ly1, lx2, ly2, lpk, cnthist, basearr, best2, best, sem):
    wid = lax.axis_index("s") * 2 + lax.axis_index("c")
    dlo = jnp.minimum(wid * DCH, NDET - DCH)

    descs = [
        pltpu.async_copy(sx1_h, sx1, sem),
        pltpu.async_copy(sy1_h, sy1, sem),
        pltpu.async_copy(sx2_h, sx2, sem),
        pltpu.async_copy(sy2_h, sy2, sem),
        pltpu.async_copy(dbt_h.at[pl.ds(dlo, DCH)], vx1, sem),
        pltpu.async_copy(dbt_h.at[pl.ds(NDET + dlo, DCH)], vy1, sem),
        pltpu.async_copy(dbt_h.at[pl.ds(2 * NDET + dlo, DCH)], vx2, sem),
        pltpu.async_copy(dbt_h.at[pl.ds(3 * NDET + dlo, DCH)], vy2, sem),
        pltpu.async_copy(dlab_h.at[pl.ds(dlo, DCH)], vlab, sem),
        pltpu.async_copy(dpk_h.at[pl.ds(dlo, DCH)], vpk, sem),
    ]

    zeros16 = jnp.zeros((16,), jnp.float32)
    zeros16i = jnp.zeros((16,), jnp.int32)

    @plsc.parallel_loop(0, 16 * RS // 16, 1, unroll=4)
    def zbody(i):
        best2[pl.ds(i * 16, 16)] = zeros16

    @plsc.parallel_loop(0, LP // 16, 1, unroll=4)
    def zlbody(i):
        lpk[pl.ds(i * 16, 16)] = zeros16i

    cnthist[pl.ds(0, 16)] = zeros16i
    cnthist[pl.ds(16, 16)] = zeros16i

    for d in descs:
        d.wait()

    def h1body(g, carry):
        lv = vlab[pl.ds(g * 16, 16)]
        scan, lmask = plsc.scan_count(lv)
        cur = plsc.load_gather(cnthist, [lv])
        plsc.store_scatter(cnthist, [lv], cur + scan, mask=lmask)
        return carry
    lax.fori_loop(0, DCH // 16, h1body, 0)

    h0 = cnthist[pl.ds(0, 16)]
    h1 = cnthist[pl.ds(16, 16)]
    ha0 = ((h0 + 15) >> 4) << 4
    ha1 = ((h1 + 15) >> 4) << 4
    e0 = plsc.cumsum(ha0) - ha0
    tot0 = jnp.sum(ha0)
    e1 = plsc.cumsum(ha1) - ha1 + tot0
    basearr[pl.ds(0, 16)] = e0
    basearr[pl.ds(16, 16)] = e1

    cnthist[pl.ds(0, 16)] = zeros16i
    cnthist[pl.ds(16, 16)] = zeros16i

    def s2body(g, carry):
        o = g * 16
        lv = vlab[pl.ds(o, 16)]
        scan, lmask = plsc.scan_count(lv)
        cur = plsc.load_gather(cnthist, [lv])
        bse = plsc.load_gather(basearr, [lv])
        pos = bse + cur + scan - 1
        plsc.store_scatter(lx1, [pos], vx1[pl.ds(o, 16)])
        plsc.store_scatter(ly1, [pos], vy1[pl.ds(o, 16)])
        plsc.store_scatter(lx2, [pos], vx2[pl.ds(o, 16)])
        plsc.store_scatter(ly2, [pos], vy2[pl.ds(o, 16)])
        plsc.store_scatter(lpk, [pos], vpk[pl.ds(o, 16)])
        plsc.store_scatter(cnthist, [lv], cur + scan, mask=lmask)
        return carry
    lax.fori_loop(0, DCH // 16, s2body, 0)

    lanerow = lax.broadcasted_iota(jnp.int32, (16,), 0) * RS
    k16 = [jnp.full((16,), k, jnp.int32) for k in range(16)]

    def gbody(g, carry):
        b16 = g * 16
        cx1 = lx1[pl.ds(b16, 16)]
        cy1 = ly1[pl.ds(b16, 16)]
        cx2 = lx2[pl.ds(b16, 16)]
        cy2 = ly2[pl.ds(b16, 16)]
        pk = lpk[pl.ds(b16, 16)]
        cnv = pk & 2047
        da = (cx2 - cx1) * (cy2 - cy1)
        pks = jnp.max(pk)
        bks = pks >> 11
        cs = pks & 2047
        nt = (cs + 15) >> 4

        @plsc.parallel_loop(0, nt, 1, unroll=1)
        def wbody(tt):
            o = bks + tt * 16
            wx1 = sx1[pl.ds(o, 16)]
            wy1 = sy1[pl.ds(o, 16)]
            wx2 = sx2[pl.ds(o, 16)]
            wy2 = sy2[pl.ds(o, 16)]
            for k in range(16):
                t = tt * 16 + k
                m = cnv > t
                g1x = jnp.take(wx1, k16[k])
                g1y = jnp.take(wy1, k16[k])
                g2x = jnp.take(wx2, k16[k])
                g2y = jnp.take(wy2, k16[k])
                ga = (g2x - g1x) * (g2y - g1y)
                lox = jnp.maximum(g1x, cx1)
                loy = jnp.maximum(g1y, cy1)
                hix = jnp.minimum(g2x, cx2)
                hiy = jnp.minimum(g2y, cy2)
                ww = jnp.maximum(hix - lox, 0.0)
                hh = jnp.maximum(hiy - loy, 0.0)
                inter = ww * hh
                uni = ga + da - inter
                iou = inter / uni
                bidx = lanerow + (o + k)
                cur = plsc.load_gather(best2, [bidx], mask=m)
                plsc.store_scatter(best2, [bidx], jnp.maximum(cur, iou), mask=m)
        return carry

    lax.fori_loop(0, LP // 16, gbody, 0)

    @plsc.parallel_loop(0, P // 16, 1, unroll=2)
    def rbody(i):
        o = i * 16
        acc = best2[pl.ds(o, 16)]
        for r in range(1, 16):
            acc = jnp.maximum(acc, best2[pl.ds(r * RS + o, 16)])
        best[pl.ds(o, 16)] = acc

    pltpu.sync_copy(best, out_h.at[wid])


def _fin_body(parts_ref, nmat_ref, out_ref):
    parts = parts_ref[...]
    best = jnp.max(parts, axis=0)
    s = jnp.sum(best)
    n = nmat_ref[0, 0]
    out_ref[0, 0] = (n - s) / n


def kernel(det_boxes, det_scores, det_labels, boxes, labels):
    del det_scores
    db = det_boxes[0]
    dl = det_labels[0].astype(jnp.int32)
    gb = boxes[0]
    gl = labels[0].astype(jnp.int32)

    gbt_gt = gb.T

    dlp = jnp.full((DP,), -1, jnp.int32).at[:NDET].set(dl)
    dbt = db.T.reshape(4 * NDET)

    _vmem = pl.BlockSpec(memory_space=pltpu.VMEM)
    _smem = pl.BlockSpec(memory_space=pltpu.SMEM)
    prep = pl.pallas_call(
        _prep_body,
        out_shape=[
            jax.ShapeDtypeStruct((P, 1), jnp.float32),
            jax.ShapeDtypeStruct((P, 1), jnp.float32),
            jax.ShapeDtypeStruct((P, 1), jnp.float32),
            jax.ShapeDtypeStruct((P, 1), jnp.float32),
            jax.ShapeDtypeStruct((DP // 128, 128), jnp.int32),
            jax.ShapeDtypeStruct((1, 1), jnp.float32),
        ],
        out_specs=[_vmem] * 5 + [_smem],
    )(gl.reshape(OP, 1), gl.reshape(1, OP), gbt_gt,
      dlp.reshape(DP // 128, 128))
    sx1, sy1, sx2, sy2, dpk, nmat = prep

    mesh = plsc.VectorSubcoreMesh(core_axis_name="c", subcore_axis_name="s")
    sc_main = functools.partial(
        pl.kernel,
        out_type=jax.ShapeDtypeStruct((NW, P), jnp.float32),
        mesh=mesh,
        compiler_params=pltpu.CompilerParams(needs_layout_passes=False),
        scratch_types=[
            pltpu.VMEM((P,), jnp.float32),
            pltpu.VMEM((P,), jnp.float32),
            pltpu.VMEM((P,), jnp.float32),
            pltpu.VMEM((P,), jnp.float32),
            pltpu.VMEM((DCH,), jnp.float32),
            pltpu.VMEM((DCH,), jnp.float32),
            pltpu.VMEM((DCH,), jnp.float32),
            pltpu.VMEM((DCH,), jnp.float32),
            pltpu.VMEM((DCH,), jnp.int32),
            pltpu.VMEM((DCH,), jnp.int32),
            pltpu.VMEM((LP,), jnp.float32),
            pltpu.VMEM((LP,), jnp.float32),
            pltpu.VMEM((LP,), jnp.float32),
            pltpu.VMEM((LP,), jnp.float32),
            pltpu.VMEM((LP,), jnp.int32),
            pltpu.VMEM((32,), jnp.int32),
            pltpu.VMEM((32,), jnp.int32),
            pltpu.VMEM((16 * RS,), jnp.float32),
            pltpu.VMEM((P,), jnp.float32),
            pltpu.SemaphoreType.DMA,
        ],
    )(_sc_body)
    parts = sc_main(sx1.reshape(P), sy1.reshape(P), sx2.reshape(P), sy2.reshape(P),
                    dbt, dl, dpk.reshape(DP))

    loss = pl.pallas_call(
        _fin_body,
        out_shape=jax.ShapeDtypeStruct((1, 1), jnp.float32),
        in_specs=[_vmem, _smem],
        out_specs=_smem,
    )(parts, nmat)
    return loss.reshape(())
